# baseline (speedup 1.0000x reference)
import jax
import jax.numpy as jnp
from jax import lax
from jax.experimental import pallas as pl
from jax.experimental.pallas import tpu as pltpu

_BM = 2048


def _phase1_body(n, x_ref, w_ref, sraw_ref, csum_ref, acc_ref):
    i = pl.program_id(0)

    @pl.when(i == 0)
    def _init():
        acc_ref[0] = 0.0

    xv = x_ref[...]
    w = w_ref[...]
    w2 = w * lax.rsqrt(jnp.sum(w * w))
    s_row = lax.dot_general(
        w2, xv, (((1,), (1,)), ((), ())), preferred_element_type=jnp.float32
    )
    col = lax.broadcasted_iota(jnp.int32, (1, _BM), 1) + i * _BM
    s_row = jnp.where(col < n, s_row, 0.0)
    sraw_ref[...] = s_row
    acc_ref[0] += jnp.sum(s_row)
    csum_ref[...] = jnp.full((1, 1), acc_ref[0], jnp.float32)


def _phase2_body(n, x_ref, w_ref, sraw_ref, csum_ref, xout_ref, score_ref):
    xv = x_ref[...]
    w = w_ref[...]
    d = w.shape[1]
    c = csum_ref[0, 0] / n
    w2t = lax.transpose(w * lax.rsqrt(jnp.sum(w * w)), (1, 0))
    wb = lax.broadcast_in_dim(w2t, (d, d), (0, 1))
    sb = lax.dot_general(
        xv, wb, (((1,), (0,)), ((), ())), preferred_element_type=jnp.float32
    )
    xout_ref[...] = xv * jnp.tanh(sb - c)
    score_ref[...] = jnp.tanh(sraw_ref[...] - c)


def kernel(x, edge_index, weight):
    n, d = x.shape
    nb = (n + _BM - 1) // _BM

    def body1(*refs):
        _phase1_body(n, *refs)

    s_raw, csum = pl.pallas_call(
        body1,
        grid=(nb,),
        in_specs=[
            pl.BlockSpec((_BM, d), lambda i: (i, 0)),
            pl.BlockSpec((1, d), lambda i: (0, 0)),
        ],
        out_specs=[
            pl.BlockSpec((1, _BM), lambda i: (0, i)),
            pl.BlockSpec((1, 1), lambda i: (0, 0)),
        ],
        out_shape=(
            jax.ShapeDtypeStruct((1, n), x.dtype),
            jax.ShapeDtypeStruct((1, 1), jnp.float32),
        ),
        scratch_shapes=[pltpu.SMEM((1,), jnp.float32)],
    )(x, weight)

    def body2(*refs):
        _phase2_body(n, *refs)

    x_out, score = pl.pallas_call(
        body2,
        grid=(nb,),
        in_specs=[
            pl.BlockSpec((_BM, d), lambda i: (i, 0)),
            pl.BlockSpec((1, d), lambda i: (0, 0)),
            pl.BlockSpec((1, _BM), lambda i: (0, i)),
            pl.BlockSpec((1, 1), lambda i: (0, 0)),
        ],
        out_specs=[
            pl.BlockSpec((_BM, d), lambda i: (i, 0)),
            pl.BlockSpec((1, _BM), lambda i: (0, i)),
        ],
        out_shape=(
            jax.ShapeDtypeStruct((n, d), x.dtype),
            jax.ShapeDtypeStruct((1, n), x.dtype),
        ),
    )(x, weight, s_raw, csum)
    return x_out, score


# speedup vs baseline: 2.0672x; 2.0672x over previous
import jax
import jax.numpy as jnp
from jax import lax
from jax.experimental import pallas as pl


def _body(n, x_ref, w_ref, xout_ref, score_ref):
    xv = x_ref[...]
    w = w_ref[...]
    d = w.shape[1]
    w2 = w * lax.rsqrt(jnp.sum(w * w))
    s_row = lax.dot_general(
        w2, xv, (((1,), (1,)), ((), ())), preferred_element_type=jnp.float32
    )
    c = jnp.sum(s_row) / n
    score_ref[...] = jnp.tanh(s_row - c)
    w2t = lax.transpose(w2, (1, 0))
    wb = lax.broadcast_in_dim(w2t, (d, d), (0, 1))
    sb = lax.dot_general(
        xv, wb, (((1,), (0,)), ((), ())), preferred_element_type=jnp.float32
    )
    xout_ref[...] = xv * jnp.tanh(sb - c)


def kernel(x, edge_index, weight):
    n, d = x.shape

    def body(*refs):
        _body(n, *refs)

    x_out, score = pl.pallas_call(
        body,
        out_shape=(
            jax.ShapeDtypeStruct((n, d), x.dtype),
            jax.ShapeDtypeStruct((1, n), x.dtype),
        ),
    )(x, weight)
    return x_out, score
